# baseline (speedup 1.0000x reference)
import functools

import jax
import jax.numpy as jnp
from jax import lax
from jax.experimental import pallas as pl
from jax.experimental.pallas import tpu as pltpu
from jax.experimental.pallas import tpu_sc as plsc

DIM = 64
CHUNK = 128
NBUF = 4


@functools.cache
def _build(n_b: int, n_t: int, vocab: int):
    info = plsc.get_sparse_core_info()
    nc = info.num_cores
    nw = nc * info.num_subcores
    b_per_w = n_b // nw
    cpt = b_per_w // CHUNK
    n_chunks = n_t * cpt
    n_outer = n_chunks // NBUF
    assert b_per_w % CHUNK == 0 and n_chunks % NBUF == 0

    mesh = plsc.VectorSubcoreMesh(core_axis_name="c", subcore_axis_name="s")

    @functools.partial(
        pl.kernel,
        mesh=mesh,
        out_type=jax.ShapeDtypeStruct((n_t, DIM // 8, n_b // 128, 8, 128), jnp.float32),
        scratch_types=[
            pltpu.VMEM((n_t, b_per_w), jnp.int32),
            pltpu.VMEM((NBUF, CHUNK, DIM), jnp.float32),
            pltpu.VMEM((DIM // 8, 8, 128), jnp.float32),
            [pltpu.SemaphoreType.DMA] * NBUF,
        ],
        compiler_params=pltpu.CompilerParams(
            use_tc_tiling_on_sc=False, needs_layout_passes=False
        ),
    )
    def gather_kernel(ids_hbm, table_hbm, out_hbm, idx_v, rows_v, tbuf, gsems):
        wid = lax.axis_index("s") * nc + lax.axis_index("c")
        b0 = wid * b_per_w
        pltpu.sync_copy(ids_hbm.at[:, pl.ds(b0, b_per_w)], idx_v)

        def fire(g, b):
            t = g // cpt
            j = g - t * cpt
            pltpu.async_copy(
                table_hbm.at[idx_v.at[t].at[pl.ds(j * CHUNK, CHUNK)]],
                rows_v.at[b],
                gsems[b],
            )

        lane = lax.iota(jnp.int32, 16)

        def drain_write(g, b):
            t = g // cpt
            j = g - t * cpt
            pltpu.make_async_copy(
                table_hbm.at[idx_v.at[t].at[pl.ds(j * CHUNK, CHUNK)]],
                rows_v.at[b],
                gsems[b],
            ).wait()

            def tpose(cc, carry):
                col = jnp.zeros((16,), jnp.int32) + cc
                for bg in range(CHUNK // 16):
                    v = plsc.load_gather(rows_v.at[b], [bg * 16 + lane, col])
                    tbuf.at[cc // 8].at[cc % 8][pl.ds(bg * 16, 16)] = v
                return carry

            lax.fori_loop(0, DIM, tpose, 0)
            bb = (b0 + j * CHUNK) // 128
            pltpu.sync_copy(tbuf, out_hbm.at[t, :, bb])

        for b in range(NBUF):
            fire(b, b)

        def outer(i, carry):
            for b in range(NBUF):
                g = i * NBUF + b
                drain_write(g, b)
                fire(g + NBUF, b)
            return carry

        lax.fori_loop(0, n_outer - 1, outer, 0)
        for b in range(NBUF):
            drain_write((n_outer - 1) * NBUF + b, b)

    return gather_kernel


def kernel(ids, emb_matrix):
    n_b, n_t = ids.shape
    vocab, dim = emb_matrix.shape
    ids_t = jnp.swapaxes(ids, 0, 1).astype(jnp.int32)
    out5 = _build(n_b, n_t, vocab)(ids_t, emb_matrix)
    return out5.transpose(2, 4, 0, 1, 3).reshape(n_b, n_t, dim)
